# baseline (speedup 1.0000x reference)
import functools

import jax
import jax.numpy as jnp
from jax import lax
from jax.experimental import pallas as pl
from jax.experimental.pallas import tpu as pltpu
from jax.experimental.pallas import tpu_sc as plsc

_VOCAB = 1000000
_D = 64
_B = 16384
_NC = 2
_NS = 16
_NW = _NC * _NS
_BPW = _B // _NW
_CHUNK = 128
_NCHUNK = _BPW // _CHUNK


def _gather_body(word_hbm, ctx_hbm, emb_hbm, ctxtab_hbm, u_hbm, v_hbm,
                 idx_u, idx_v, rows_u, rows_v, sem_u, sem_v):
    wid = lax.axis_index("s") * _NC + lax.axis_index("c")
    rbase = wid * _NCHUNK
    base = wid * _BPW
    pltpu.sync_copy(word_hbm.at[pl.ds(rbase, _NCHUNK)], idx_u)
    pltpu.sync_copy(ctx_hbm.at[pl.ds(rbase, _NCHUNK)], idx_v)
    cps = []
    for j in range(_NCHUNK):
        cps.append(pltpu.async_copy(
            emb_hbm.at[idx_u.at[j]],
            rows_u.at[pl.ds(j * _CHUNK, _CHUNK)], sem_u))
        cps.append(pltpu.async_copy(
            ctxtab_hbm.at[idx_v.at[j]],
            rows_v.at[pl.ds(j * _CHUNK, _CHUNK)], sem_v))
    for cp in cps:
        cp.wait()
    pltpu.sync_copy(rows_u, u_hbm.at[pl.ds(base, _BPW)])
    pltpu.sync_copy(rows_v, v_hbm.at[pl.ds(base, _BPW)])


def _loss_body(u_ref, v_ref, loss_ref):
    p = u_ref[...] * v_ref[...]
    s = jnp.sum(p, axis=1)
    ls = jnp.minimum(s, 0.0) - jnp.log1p(jnp.exp(-jnp.abs(s)))
    loss_ref[0, 0] = -jnp.sum(ls) * (1.0 / _B)


@jax.jit
def kernel(word, context, emb_table, ctx_table):
    word2 = word.reshape(_B // _CHUNK, _CHUNK)
    ctx2 = context.reshape(_B // _CHUNK, _CHUNK)
    mesh = plsc.VectorSubcoreMesh(core_axis_name="c", subcore_axis_name="s")
    gather = pl.kernel(
        _gather_body,
        out_type=[jax.ShapeDtypeStruct((_B, _D), jnp.float32),
                  jax.ShapeDtypeStruct((_B, _D), jnp.float32)],
        mesh=mesh,
        scratch_types=[
            pltpu.VMEM((_NCHUNK, _CHUNK), jnp.int32),
            pltpu.VMEM((_NCHUNK, _CHUNK), jnp.int32),
            pltpu.VMEM((_BPW, _D), jnp.float32),
            pltpu.VMEM((_BPW, _D), jnp.float32),
            pltpu.SemaphoreType.DMA,
            pltpu.SemaphoreType.DMA,
        ],
        compiler_params=pltpu.CompilerParams(use_tc_tiling_on_sc=False),
    )
    embed_u, embed_v = gather(word2, ctx2, emb_table, ctx_table)
    loss2 = pl.pallas_call(
        _loss_body,
        out_shape=jax.ShapeDtypeStruct((1, 1), jnp.float32),
        out_specs=pl.BlockSpec(memory_space=pltpu.SMEM),
    )(embed_u, embed_v)
    return loss2[0, 0], embed_u


# speedup vs baseline: 1.5793x; 1.5793x over previous
import functools

import jax
import jax.numpy as jnp
from jax import lax
from jax.experimental import pallas as pl
from jax.experimental.pallas import tpu as pltpu
from jax.experimental.pallas import tpu_sc as plsc

_VOCAB = 1000000
_D = 64
_B = 16384
_NC = 2
_NS = 16
_NW = _NC * _NS
_BPW = _B // _NW
_HB = 256


def _gather_body(word_hbm, ctx_hbm, emb_hbm, ctxtab_hbm, u_hbm, v_hbm,
                 idx_u, idx_v, rows_u, rows_v, sem_u, sem_v, sem_i):
    wid = lax.axis_index("s") * _NC + lax.axis_index("c")
    base = wid * _BPW
    pltpu.async_copy(word_hbm.at[pl.ds(base, _BPW)], idx_u, sem_i).wait()
    pltpu.async_copy(ctx_hbm.at[pl.ds(base, _BPW)], idx_v, sem_i).wait()

    for h in range(_BPW // _HB):
        hb = h * _HB

        @pl.loop(0, _HB, step=16)
        def _(r):
            iu = idx_u[pl.ds(hb + r, 16)]
            iv = idx_v[pl.ds(hb + r, 16)]
            for j in range(16):
                pltpu.async_copy(emb_hbm.at[pl.ds(iu[j], 1)],
                                 rows_u.at[pl.ds(r + j, 1)], sem_u)
                pltpu.async_copy(ctxtab_hbm.at[pl.ds(iv[j], 1)],
                                 rows_v.at[pl.ds(r + j, 1)], sem_v)

        pltpu.make_async_copy(emb_hbm.at[pl.ds(0, _HB)], rows_u, sem_u).wait()
        pltpu.make_async_copy(ctxtab_hbm.at[pl.ds(0, _HB)], rows_v,
                              sem_v).wait()

        pltpu.sync_copy(rows_u, u_hbm.at[pl.ds(base + hb, _HB)])
        pltpu.sync_copy(rows_v, v_hbm.at[pl.ds(base + hb, _HB)])


def _loss_body(u_ref, v_ref, loss_ref):
    p = u_ref[...] * v_ref[...]
    s = jnp.sum(p, axis=1)
    ls = jnp.minimum(s, 0.0) - jnp.log1p(jnp.exp(-jnp.abs(s)))
    loss_ref[0, 0] = -jnp.sum(ls) * (1.0 / _B)


@jax.jit
def kernel(word, context, emb_table, ctx_table):
    mesh = plsc.VectorSubcoreMesh(core_axis_name="c", subcore_axis_name="s")
    gather = pl.kernel(
        _gather_body,
        out_type=[jax.ShapeDtypeStruct((_B, _D), jnp.float32),
                  jax.ShapeDtypeStruct((_B, _D), jnp.float32)],
        mesh=mesh,
        scratch_types=[
            pltpu.VMEM((_BPW,), jnp.int32),
            pltpu.VMEM((_BPW,), jnp.int32),
            pltpu.VMEM((_HB, _D), jnp.float32),
            pltpu.VMEM((_HB, _D), jnp.float32),
            pltpu.SemaphoreType.DMA,
            pltpu.SemaphoreType.DMA,
            pltpu.SemaphoreType.DMA,
        ],
        compiler_params=pltpu.CompilerParams(use_tc_tiling_on_sc=True),
    )
    embed_u, embed_v = gather(word, context, emb_table, ctx_table)
    loss2 = pl.pallas_call(
        _loss_body,
        out_shape=jax.ShapeDtypeStruct((1, 1), jnp.float32),
        out_specs=pl.BlockSpec(memory_space=pltpu.SMEM),
    )(embed_u, embed_v)
    return loss2[0, 0], embed_u
